# Initial kernel scaffold; baseline (speedup 1.0000x reference)
import jax
import jax.numpy as jnp
from jax.experimental import pallas as pl


def kernel(x, edge_index, W1_l, W1_r, b1, W2_l, W2_r, b2):
    raise NotImplementedError("write your pallas kernel here")


# speedup vs baseline: 6.1513x; 6.1513x over previous
import functools

import jax
import jax.numpy as jnp
from jax import lax
from jax.experimental import pallas as pl
from jax.experimental.pallas import tpu as pltpu
from jax.experimental.pallas import tpu_sc as plsc

N = 10000
E = 320000
D_IN = 128
D_HID = 256
D_OUT = 100

NC = 2
NS = 16
NW = NC * NS

K = 80
EPW = E // NW
NCHUNK = EPW // K
ROWS_MAIN = 640
ROWS_TAIL = N - (NS - 1) * ROWS_MAIN


def _segsum_body(table_hbm, src_hbm, dst_hbm, zrows_hbm, zn_hbm,
                 out_hbm, cnt_hbm,
                 acc_sh, cnt_sh, src_v, dst_v, rows_v, ones_v, sem):
  c = lax.axis_index("c")
  s = lax.axis_index("s")
  wid = c * NS + s

  for i in range(K // 16):
    ones_v[pl.ds(i * 16, 16)] = jnp.ones((16,), jnp.float32)

  r0 = s * ROWS_MAIN

  @pl.when(s < NS - 1)
  def _():
    pltpu.sync_copy(zrows_hbm.at[pl.ds(r0, ROWS_MAIN)],
                    acc_sh.at[pl.ds(r0, ROWS_MAIN)])

  @pl.when(s == NS - 1)
  def _():
    pltpu.sync_copy(zrows_hbm.at[pl.ds(r0, ROWS_TAIL)],
                    acc_sh.at[pl.ds(r0, ROWS_TAIL)])

  @pl.when(s == 0)
  def _():
    pltpu.sync_copy(zn_hbm, cnt_sh)

  plsc.subcore_barrier()

  def body(j, carry):
    start = wid * EPW + j * K
    pltpu.sync_copy(src_hbm.at[pl.ds(start, K)], src_v)
    pltpu.sync_copy(dst_hbm.at[pl.ds(start, K)], dst_v)
    pltpu.async_copy(table_hbm.at[src_v], rows_v, sem).wait()
    pltpu.sync_copy(rows_v, acc_sh.at[dst_v], add=True)
    pltpu.sync_copy(ones_v, cnt_sh.at[dst_v], add=True)
    return carry

  lax.fori_loop(0, NCHUNK, body, 0)

  plsc.subcore_barrier()

  @pl.when(s < NS - 1)
  def _():
    pltpu.sync_copy(acc_sh.at[pl.ds(r0, ROWS_MAIN)],
                    out_hbm.at[c, pl.ds(r0, ROWS_MAIN)])

  @pl.when(s == NS - 1)
  def _():
    pltpu.sync_copy(acc_sh.at[pl.ds(r0, ROWS_TAIL)],
                    out_hbm.at[c, pl.ds(r0, ROWS_TAIL)])

  @pl.when(s == 0)
  def _():
    pltpu.sync_copy(cnt_sh, cnt_hbm.at[c])


def _segment_sum_sc(table, src, dst):
  zrows = jnp.zeros((N, D_IN), jnp.float32)
  zn = jnp.zeros((N,), jnp.float32)
  mesh = plsc.VectorSubcoreMesh(core_axis_name="c", subcore_axis_name="s",
                                num_cores=NC, num_subcores=NS)
  f = pl.kernel(
      _segsum_body,
      out_type=(jax.ShapeDtypeStruct((NC, N, D_IN), jnp.float32),
                jax.ShapeDtypeStruct((NC, N), jnp.float32)),
      mesh=mesh,
      scratch_types=[
          pltpu.VMEM_SHARED((N, D_IN), jnp.float32),
          pltpu.VMEM_SHARED((N,), jnp.float32),
          pltpu.VMEM((K,), jnp.int32),
          pltpu.VMEM((K,), jnp.int32),
          pltpu.VMEM((K, D_IN), jnp.float32),
          pltpu.VMEM((K,), jnp.float32),
          pltpu.SemaphoreType.DMA,
      ],
  )
  return f(table, src, dst, zrows, zn)


BN = 1000


def _layer1_tc_body(accA, accB, cntA, cntB, x, w1l, w1r, b1, w2lp,
                    h_ref, g_ref, rc_ref):
  cnt = jnp.maximum(cntA[...] + cntB[...], 1.0)
  rc = 1.0 / cnt
  agg = (accA[...] + accB[...]) * rc
  h = (jnp.dot(agg, w1l[...], preferred_element_type=jnp.float32)
       + b1[...]
       + jnp.dot(x[...], w1r[...], preferred_element_type=jnp.float32))
  h = jnp.maximum(h, 0.0)
  h_ref[...] = h
  g_ref[...] = jnp.dot(h, w2lp[...], preferred_element_type=jnp.float32)
  rc_ref[...] = rc


def _layer2_tc_body(accA, accB, rc, h, w2rp, b2p, o_ref):
  z = ((accA[...] + accB[...]) * rc[...]
       + jnp.dot(h[...], w2rp[...], preferred_element_type=jnp.float32)
       + b2p[...])
  o_ref[...] = jax.nn.sigmoid(z)


def _row_spec(d):
  return pl.BlockSpec((BN, d), lambda i: (i, 0))


def _full_spec(r, c):
  return pl.BlockSpec((r, c), lambda i: (0, 0))


def kernel(x, edge_index, W1_l, W1_r, b1, W2_l, W2_r, b2):
  src = edge_index[0]
  dst = edge_index[1]

  acc1, cnt = _segment_sum_sc(x, src, dst)

  w2lp = jnp.pad(W2_l, ((0, 0), (0, D_IN - D_OUT)))
  grid = (N // BN,)
  h, g, rc = pl.pallas_call(
      _layer1_tc_body,
      grid=grid,
      in_specs=[
          _row_spec(D_IN), _row_spec(D_IN),
          _row_spec(1), _row_spec(1),
          _row_spec(D_IN),
          _full_spec(D_IN, D_HID), _full_spec(D_IN, D_HID),
          _full_spec(1, D_HID),
          _full_spec(D_HID, D_IN),
      ],
      out_specs=[_row_spec(D_HID), _row_spec(D_IN), _row_spec(1)],
      out_shape=[
          jax.ShapeDtypeStruct((N, D_HID), jnp.float32),
          jax.ShapeDtypeStruct((N, D_IN), jnp.float32),
          jax.ShapeDtypeStruct((N, 1), jnp.float32),
      ],
  )(acc1[0], acc1[1], cnt[0][:, None], cnt[1][:, None], x,
    W1_l, W1_r, b1[None, :], w2lp)

  acc2, _ = _segment_sum_sc(g, src, dst)

  w2rp = jnp.pad(W2_r, ((0, 0), (0, D_IN - D_OUT)))
  b2p = jnp.pad(b2, (0, D_IN - D_OUT))
  o = pl.pallas_call(
      _layer2_tc_body,
      grid=grid,
      in_specs=[
          _row_spec(D_IN), _row_spec(D_IN),
          _row_spec(1),
          _row_spec(D_HID),
          _full_spec(D_HID, D_IN),
          _full_spec(1, D_IN),
      ],
      out_specs=_row_spec(D_IN),
      out_shape=jax.ShapeDtypeStruct((N, D_IN), jnp.float32),
  )(acc2[0], acc2[1], rc, h, w2rp, b2p[None, :])

  return o[:, :D_OUT]


# speedup vs baseline: 11.6031x; 1.8863x over previous
import functools

import jax
import jax.numpy as jnp
from jax import lax
from jax.experimental import pallas as pl
from jax.experimental.pallas import tpu as pltpu
from jax.experimental.pallas import tpu_sc as plsc

N = 10000
E = 320000
D_IN = 128
D_HID = 256
D_OUT = 100

NC = 2
NS = 16
NW = NC * NS

K = 80
EPW = E // NW
NCHUNK = EPW // K
ROWS_MAIN = 640
ROWS_TAIL = N - (NS - 1) * ROWS_MAIN


def _segsum_body(with_cnt, table_hbm, idx_hbm, zrows_hbm, zn_hbm,
                 out_hbm, cnt_hbm,
                 acc_sh, cnt_sh, idx0, idx1, rows0, rows1, ones_v,
                 sem0, sem1):
  c = lax.axis_index("c")
  s = lax.axis_index("s")
  wid = c * NS + s

  if with_cnt:
    for i in range(K // 16):
      ones_v[pl.ds(i * 16, 16)] = jnp.ones((16,), jnp.float32)

  r0 = s * ROWS_MAIN

  @pl.when(s < NS - 1)
  def _():
    pltpu.sync_copy(zrows_hbm.at[pl.ds(r0, ROWS_MAIN)],
                    acc_sh.at[pl.ds(r0, ROWS_MAIN)])

  @pl.when(s == NS - 1)
  def _():
    pltpu.sync_copy(zrows_hbm.at[pl.ds(r0, ROWS_TAIL)],
                    acc_sh.at[pl.ds(r0, ROWS_TAIL)])

  if with_cnt:
    @pl.when(s == 0)
    def _():
      pltpu.sync_copy(zn_hbm, cnt_sh)

  plsc.subcore_barrier()

  def load_idx(idx_v, j):
    pltpu.sync_copy(idx_hbm.at[wid, j], idx_v)

  def gather(idx_v, rows, sem):
    pltpu.async_copy(table_hbm.at[idx_v.at[0]], rows, sem)

  def wait(rows, sem):
    pltpu.make_async_copy(table_hbm.at[idx0.at[0]], rows, sem).wait()

  def scatter(rows, idx_v):
    pltpu.sync_copy(rows, acc_sh.at[idx_v.at[1]], add=True)
    if with_cnt:
      pltpu.sync_copy(ones_v, cnt_sh.at[idx_v.at[1]], add=True)

  load_idx(idx0, 0)
  gather(idx0, rows0, sem0)

  def body(i, carry):
    j = 2 * i + 1
    load_idx(idx1, j)
    gather(idx1, rows1, sem1)
    wait(rows0, sem0)
    scatter(rows0, idx0)
    load_idx(idx0, j + 1)
    gather(idx0, rows0, sem0)
    wait(rows1, sem1)
    scatter(rows1, idx1)
    return carry

  lax.fori_loop(0, (NCHUNK - 1) // 2, body, 0)
  wait(rows0, sem0)
  scatter(rows0, idx0)

  plsc.subcore_barrier()

  @pl.when(s < NS - 1)
  def _():
    pltpu.sync_copy(acc_sh.at[pl.ds(r0, ROWS_MAIN)],
                    out_hbm.at[c, pl.ds(r0, ROWS_MAIN)])

  @pl.when(s == NS - 1)
  def _():
    pltpu.sync_copy(acc_sh.at[pl.ds(r0, ROWS_TAIL)],
                    out_hbm.at[c, pl.ds(r0, ROWS_TAIL)])

  if with_cnt:
    @pl.when(s == 0)
    def _():
      pltpu.sync_copy(cnt_sh, cnt_hbm.at[c])


def _segment_sum_sc(table, idx, with_cnt):
  zrows = jnp.zeros((N, D_IN), jnp.float32)
  zn = jnp.zeros((N,), jnp.float32)
  mesh = plsc.VectorSubcoreMesh(core_axis_name="c", subcore_axis_name="s",
                                num_cores=NC, num_subcores=NS)
  f = pl.kernel(
      functools.partial(_segsum_body, with_cnt),
      out_type=(jax.ShapeDtypeStruct((NC, N, D_IN), jnp.float32),
                jax.ShapeDtypeStruct((NC, N), jnp.float32)),
      mesh=mesh,
      scratch_types=[
          pltpu.VMEM_SHARED((N, D_IN), jnp.float32),
          pltpu.VMEM_SHARED((N,), jnp.float32),
          pltpu.VMEM((2, K), jnp.int32),
          pltpu.VMEM((2, K), jnp.int32),
          pltpu.VMEM((K, D_IN), jnp.float32),
          pltpu.VMEM((K, D_IN), jnp.float32),
          pltpu.VMEM((K,), jnp.float32),
          pltpu.SemaphoreType.DMA,
          pltpu.SemaphoreType.DMA,
      ],
  )
  return f(table, idx, zrows, zn)


BN = 1000


def _layer1_tc_body(accA, accB, cntA, cntB, x, w1l, w1r, b1, w2lp,
                    h_ref, g_ref, rc_ref):
  cnt = jnp.maximum(cntA[...] + cntB[...], 1.0)
  rc = 1.0 / cnt
  agg = (accA[...] + accB[...]) * rc
  h = (jnp.dot(agg, w1l[...], preferred_element_type=jnp.float32)
       + b1[...]
       + jnp.dot(x[...], w1r[...], preferred_element_type=jnp.float32))
  h = jnp.maximum(h, 0.0)
  h_ref[...] = h
  g_ref[...] = jnp.dot(h, w2lp[...], preferred_element_type=jnp.float32)
  rc_ref[...] = rc


def _layer2_tc_body(accA, accB, rc, h, w2rp, b2p, o_ref):
  z = ((accA[...] + accB[...]) * rc[...]
       + jnp.dot(h[...], w2rp[...], preferred_element_type=jnp.float32)
       + b2p[...])
  o_ref[...] = jax.nn.sigmoid(z)


def _row_spec(d):
  return pl.BlockSpec((BN, d), lambda i: (i, 0))


def _full_spec(r, c):
  return pl.BlockSpec((r, c), lambda i: (0, 0))


def kernel(x, edge_index, W1_l, W1_r, b1, W2_l, W2_r, b2):
  idx = jnp.transpose(edge_index.reshape(2, NW, NCHUNK, K), (1, 2, 0, 3))

  acc1, cnt = _segment_sum_sc(x, idx, with_cnt=True)

  w2lp = jnp.pad(W2_l, ((0, 0), (0, D_IN - D_OUT)))
  grid = (N // BN,)
  h, g, rc = pl.pallas_call(
      _layer1_tc_body,
      grid=grid,
      in_specs=[
          _row_spec(D_IN), _row_spec(D_IN),
          _row_spec(1), _row_spec(1),
          _row_spec(D_IN),
          _full_spec(D_IN, D_HID), _full_spec(D_IN, D_HID),
          _full_spec(1, D_HID),
          _full_spec(D_HID, D_IN),
      ],
      out_specs=[_row_spec(D_HID), _row_spec(D_IN), _row_spec(1)],
      out_shape=[
          jax.ShapeDtypeStruct((N, D_HID), jnp.float32),
          jax.ShapeDtypeStruct((N, D_IN), jnp.float32),
          jax.ShapeDtypeStruct((N, 1), jnp.float32),
      ],
  )(acc1[0], acc1[1], cnt[0][:, None], cnt[1][:, None], x,
    W1_l, W1_r, b1[None, :], w2lp)

  acc2, _ = _segment_sum_sc(g, idx, with_cnt=False)

  w2rp = jnp.pad(W2_r, ((0, 0), (0, D_IN - D_OUT)))
  b2p = jnp.pad(b2, (0, D_IN - D_OUT))
  o = pl.pallas_call(
      _layer2_tc_body,
      grid=grid,
      in_specs=[
          _row_spec(D_IN), _row_spec(D_IN),
          _row_spec(1),
          _row_spec(D_HID),
          _full_spec(D_HID, D_IN),
          _full_spec(1, D_IN),
      ],
      out_specs=_row_spec(D_IN),
      out_shape=jax.ShapeDtypeStruct((N, D_IN), jnp.float32),
  )(acc2[0], acc2[1], rc, h, w2rp, b2p[None, :])

  return o[:, :D_OUT]


# speedup vs baseline: 13.0627x; 1.1258x over previous
import functools

import jax
import jax.numpy as jnp
from jax import lax
from jax.experimental import pallas as pl
from jax.experimental.pallas import tpu as pltpu
from jax.experimental.pallas import tpu_sc as plsc

N = 10000
E = 320000
D_IN = 128
D_HID = 256
D_OUT = 100

NC = 2
NS = 16
NW = NC * NS

K = 80
EPW = E // NW
NCHUNK = EPW // K
ROWS_MAIN = 640
ROWS_TAIL = N - (NS - 1) * ROWS_MAIN


def _segsum_body(with_cnt, table_hbm, idx_hbm, zrows_hbm, zn_hbm,
                 out_hbm, cnt_hbm,
                 acc_sh, cnt_sh, idx0, idx1, idx2, rows0, rows1, rows2,
                 ones_v, gsem0, gsem1, gsem2, ssem0, ssem1, ssem2):
  c = lax.axis_index("c")
  s = lax.axis_index("s")
  wid = c * NS + s

  if with_cnt:
    for i in range(K // 16):
      ones_v[pl.ds(i * 16, 16)] = jnp.ones((16,), jnp.float32)

  r0 = s * ROWS_MAIN

  @pl.when(s < NS - 1)
  def _():
    pltpu.sync_copy(zrows_hbm.at[pl.ds(r0, ROWS_MAIN)],
                    acc_sh.at[pl.ds(r0, ROWS_MAIN)])

  @pl.when(s == NS - 1)
  def _():
    pltpu.sync_copy(zrows_hbm.at[pl.ds(r0, ROWS_TAIL)],
                    acc_sh.at[pl.ds(r0, ROWS_TAIL)])

  if with_cnt:
    @pl.when(s == 0)
    def _():
      pltpu.sync_copy(zn_hbm, cnt_sh)

  plsc.subcore_barrier()

  idx_b = (idx0, idx1, idx2)
  rows_b = (rows0, rows1, rows2)
  gsem_b = (gsem0, gsem1, gsem2)
  ssem_b = (ssem0, ssem1, ssem2)

  def load_idx(b, j):
    pltpu.sync_copy(idx_hbm.at[wid, j], idx_b[b])

  def gather(b, j):
    load_idx(b, j)
    pltpu.async_copy(table_hbm.at[idx_b[b].at[0]], rows_b[b], gsem_b[b])

  def wait_gather(b):
    pltpu.make_async_copy(table_hbm.at[idx_b[b].at[0]], rows_b[b],
                          gsem_b[b]).wait()

  def scatter_start(b):
    pltpu.make_async_copy(rows_b[b], acc_sh.at[idx_b[b].at[1]],
                          ssem_b[b]).start(add=True)
    if with_cnt:
      pltpu.sync_copy(ones_v, cnt_sh.at[idx_b[b].at[1]], add=True)

  def wait_scatter(b):
    pltpu.make_async_copy(rows_b[b], acc_sh.at[idx_b[b].at[1]],
                          ssem_b[b]).wait()

  gather(0, 0)
  gather(1, 1)

  def step(i, k):
    c = 3 * i + k
    wait_gather(k)
    scatter_start(k)
    bq = (k + 2) % 3

    def prefetch():
      wait_scatter(bq)
      gather(bq, c + 2)

    if k == 0:
      @pl.when(i > 0)
      def _():
        prefetch()

      @pl.when(i == 0)
      def _():
        gather(bq, c + 2)
    else:
      prefetch()

  def body(i, carry):
    step(i, 0)
    step(i, 1)
    step(i, 2)
    return carry

  lax.fori_loop(0, (NCHUNK - 2) // 3, body, 0)
  wait_gather(0)
  scatter_start(0)
  wait_gather(1)
  scatter_start(1)
  wait_scatter(2)
  wait_scatter(0)
  wait_scatter(1)

  plsc.subcore_barrier()

  @pl.when(s < NS - 1)
  def _():
    pltpu.sync_copy(acc_sh.at[pl.ds(r0, ROWS_MAIN)],
                    out_hbm.at[c, pl.ds(r0, ROWS_MAIN)])

  @pl.when(s == NS - 1)
  def _():
    pltpu.sync_copy(acc_sh.at[pl.ds(r0, ROWS_TAIL)],
                    out_hbm.at[c, pl.ds(r0, ROWS_TAIL)])

  if with_cnt:
    @pl.when(s == 0)
    def _():
      pltpu.sync_copy(cnt_sh, cnt_hbm.at[c])


def _segment_sum_sc(table, idx, with_cnt):
  zrows = jnp.zeros((N, D_IN), jnp.float32)
  zn = jnp.zeros((N,), jnp.float32)
  mesh = plsc.VectorSubcoreMesh(core_axis_name="c", subcore_axis_name="s",
                                num_cores=NC, num_subcores=NS)
  f = pl.kernel(
      functools.partial(_segsum_body, with_cnt),
      out_type=(jax.ShapeDtypeStruct((NC, N, D_IN), jnp.float32),
                jax.ShapeDtypeStruct((NC, N), jnp.float32)),
      mesh=mesh,
      scratch_types=[
          pltpu.VMEM_SHARED((N, D_IN), jnp.float32),
          pltpu.VMEM_SHARED((N,), jnp.float32),
          pltpu.VMEM((2, K), jnp.int32),
          pltpu.VMEM((2, K), jnp.int32),
          pltpu.VMEM((2, K), jnp.int32),
          pltpu.VMEM((K, D_IN), jnp.float32),
          pltpu.VMEM((K, D_IN), jnp.float32),
          pltpu.VMEM((K, D_IN), jnp.float32),
          pltpu.VMEM((K,), jnp.float32),
          pltpu.SemaphoreType.DMA,
          pltpu.SemaphoreType.DMA,
          pltpu.SemaphoreType.DMA,
          pltpu.SemaphoreType.DMA,
          pltpu.SemaphoreType.DMA,
          pltpu.SemaphoreType.DMA,
      ],
  )
  return f(table, idx, zrows, zn)


BN = 1000


def _layer1_tc_body(accA, accB, cntA, cntB, x, w1l, w1r, b1, w2lp,
                    h_ref, g_ref, rc_ref):
  cnt = jnp.maximum(cntA[...] + cntB[...], 1.0)
  rc = 1.0 / cnt
  agg = (accA[...] + accB[...]) * rc
  h = (jnp.dot(agg, w1l[...], preferred_element_type=jnp.float32)
       + b1[...]
       + jnp.dot(x[...], w1r[...], preferred_element_type=jnp.float32))
  h = jnp.maximum(h, 0.0)
  h_ref[...] = h
  g_ref[...] = jnp.dot(h, w2lp[...], preferred_element_type=jnp.float32)
  rc_ref[...] = rc


def _layer2_tc_body(accA, accB, rc, h, w2rp, b2p, o_ref):
  z = ((accA[...] + accB[...]) * rc[...]
       + jnp.dot(h[...], w2rp[...], preferred_element_type=jnp.float32)
       + b2p[...])
  o_ref[...] = jax.nn.sigmoid(z)


def _row_spec(d):
  return pl.BlockSpec((BN, d), lambda i: (i, 0))


def _full_spec(r, c):
  return pl.BlockSpec((r, c), lambda i: (0, 0))


def kernel(x, edge_index, W1_l, W1_r, b1, W2_l, W2_r, b2):
  idx = jnp.transpose(edge_index.reshape(2, NW, NCHUNK, K), (1, 2, 0, 3))

  acc1, cnt = _segment_sum_sc(x, idx, with_cnt=True)

  w2lp = jnp.pad(W2_l, ((0, 0), (0, D_IN - D_OUT)))
  grid = (N // BN,)
  h, g, rc = pl.pallas_call(
      _layer1_tc_body,
      grid=grid,
      in_specs=[
          _row_spec(D_IN), _row_spec(D_IN),
          _row_spec(1), _row_spec(1),
          _row_spec(D_IN),
          _full_spec(D_IN, D_HID), _full_spec(D_IN, D_HID),
          _full_spec(1, D_HID),
          _full_spec(D_HID, D_IN),
      ],
      out_specs=[_row_spec(D_HID), _row_spec(D_IN), _row_spec(1)],
      out_shape=[
          jax.ShapeDtypeStruct((N, D_HID), jnp.float32),
          jax.ShapeDtypeStruct((N, D_IN), jnp.float32),
          jax.ShapeDtypeStruct((N, 1), jnp.float32),
      ],
  )(acc1[0], acc1[1], cnt[0][:, None], cnt[1][:, None], x,
    W1_l, W1_r, b1[None, :], w2lp)

  acc2, _ = _segment_sum_sc(g, idx, with_cnt=False)

  w2rp = jnp.pad(W2_r, ((0, 0), (0, D_IN - D_OUT)))
  b2p = jnp.pad(b2, (0, D_IN - D_OUT))
  o = pl.pallas_call(
      _layer2_tc_body,
      grid=grid,
      in_specs=[
          _row_spec(D_IN), _row_spec(D_IN),
          _row_spec(1),
          _row_spec(D_HID),
          _full_spec(D_HID, D_IN),
          _full_spec(1, D_IN),
      ],
      out_specs=_row_spec(D_IN),
      out_shape=jax.ShapeDtypeStruct((N, D_IN), jnp.float32),
  )(acc2[0], acc2[1], rc, h, w2rp, b2p[None, :])

  return o[:, :D_OUT]


# speedup vs baseline: 13.6409x; 1.0443x over previous
import functools

import jax
import jax.numpy as jnp
from jax import lax
from jax.experimental import pallas as pl
from jax.experimental.pallas import tpu as pltpu
from jax.experimental.pallas import tpu_sc as plsc

N = 10000
E = 320000
D_IN = 128
D_HID = 256
D_OUT = 100

NC = 2
NS = 16
NW = NC * NS

K = 80
EPW = E // NW
NCHUNK = EPW // K
ROWS_MAIN = 640
ROWS_TAIL = N - (NS - 1) * ROWS_MAIN


def _segsum_body(with_cnt, table_hbm, idx_hbm, zrows_hbm, zn_hbm,
                 out_hbm, cnt_hbm,
                 acc_sh, cnt_sh, idx0, idx1, idx2, rows0, rows1, rows2,
                 ones_v, gsem0, gsem1, gsem2, ssem0, ssem1, ssem2):
  c = lax.axis_index("c")
  s = lax.axis_index("s")
  wid = c * NS + s

  if with_cnt:
    for i in range(K // 16):
      ones_v[pl.ds(i * 16, 16)] = jnp.ones((16,), jnp.float32)

  r0 = s * ROWS_MAIN

  @pl.when(s < NS - 1)
  def _():
    pltpu.sync_copy(zrows_hbm.at[pl.ds(r0, ROWS_MAIN)],
                    acc_sh.at[pl.ds(r0, ROWS_MAIN)])

  @pl.when(s == NS - 1)
  def _():
    pltpu.sync_copy(zrows_hbm.at[pl.ds(r0, ROWS_TAIL)],
                    acc_sh.at[pl.ds(r0, ROWS_TAIL)])

  if with_cnt:
    @pl.when(s == 0)
    def _():
      pltpu.sync_copy(zn_hbm, cnt_sh)

  plsc.subcore_barrier()

  idx_b = (idx0, idx1, idx2)
  rows_b = (rows0, rows1, rows2)
  gsem_b = (gsem0, gsem1, gsem2)
  ssem_b = (ssem0, ssem1, ssem2)

  def load_idx(b, j):
    pltpu.sync_copy(idx_hbm.at[wid, j], idx_b[b])

  def gather(b, j):
    load_idx(b, j)
    pltpu.async_copy(table_hbm.at[idx_b[b].at[0]], rows_b[b], gsem_b[b])

  def wait_gather(b):
    pltpu.make_async_copy(table_hbm.at[idx_b[b].at[0]], rows_b[b],
                          gsem_b[b]).wait()

  def scatter_start(b):
    pltpu.make_async_copy(rows_b[b], acc_sh.at[idx_b[b].at[1]],
                          ssem_b[b]).start(add=True)
    if with_cnt:
      pltpu.sync_copy(ones_v, cnt_sh.at[idx_b[b].at[1]], add=True)

  def wait_scatter(b):
    pltpu.make_async_copy(rows_b[b], acc_sh.at[idx_b[b].at[1]],
                          ssem_b[b]).wait()

  gather(0, 0)
  gather(1, 1)

  def step(i, k):
    c = 3 * i + k
    wait_gather(k)
    scatter_start(k)
    bq = (k + 2) % 3

    def prefetch():
      wait_scatter(bq)
      gather(bq, c + 2)

    if k == 0:
      @pl.when(i > 0)
      def _():
        prefetch()

      @pl.when(i == 0)
      def _():
        gather(bq, c + 2)
    else:
      prefetch()

  def body(i, carry):
    step(i, 0)
    step(i, 1)
    step(i, 2)
    return carry

  lax.fori_loop(0, (NCHUNK - 2) // 3, body, 0)
  wait_gather(0)
  scatter_start(0)
  wait_gather(1)
  scatter_start(1)
  wait_scatter(2)
  wait_scatter(0)
  wait_scatter(1)

  plsc.subcore_barrier()

  @pl.when(s < NS - 1)
  def _():
    pltpu.sync_copy(acc_sh.at[pl.ds(r0, ROWS_MAIN)],
                    out_hbm.at[c, pl.ds(r0, ROWS_MAIN)])

  @pl.when(s == NS - 1)
  def _():
    pltpu.sync_copy(acc_sh.at[pl.ds(r0, ROWS_TAIL)],
                    out_hbm.at[c, pl.ds(r0, ROWS_TAIL)])

  if with_cnt:
    @pl.when(s == 0)
    def _():
      pltpu.sync_copy(cnt_sh, cnt_hbm.at[c])


def _segment_sum_sc(table, idx, with_cnt):
  zrows = jnp.zeros((N, D_IN), jnp.float32)
  zn = jnp.zeros((N,), jnp.float32)
  mesh = plsc.VectorSubcoreMesh(core_axis_name="c", subcore_axis_name="s",
                                num_cores=NC, num_subcores=NS)
  f = pl.kernel(
      functools.partial(_segsum_body, with_cnt),
      out_type=(jax.ShapeDtypeStruct((NC, N, D_IN), jnp.float32),
                jax.ShapeDtypeStruct((NC, N), jnp.float32)),
      mesh=mesh,
      scratch_types=[
          pltpu.VMEM_SHARED((N, D_IN), jnp.float32),
          pltpu.VMEM_SHARED((N,), jnp.float32),
          pltpu.VMEM((2, K), jnp.int32),
          pltpu.VMEM((2, K), jnp.int32),
          pltpu.VMEM((2, K), jnp.int32),
          pltpu.VMEM((K, D_IN), jnp.float32),
          pltpu.VMEM((K, D_IN), jnp.float32),
          pltpu.VMEM((K, D_IN), jnp.float32),
          pltpu.VMEM((K,), jnp.float32),
          pltpu.SemaphoreType.DMA,
          pltpu.SemaphoreType.DMA,
          pltpu.SemaphoreType.DMA,
          pltpu.SemaphoreType.DMA,
          pltpu.SemaphoreType.DMA,
          pltpu.SemaphoreType.DMA,
      ],
  )
  return f(table, idx, zrows, zn)


BN = 1000


def _layer1_tc_body(acc, cnt2, x, w1l, w1r, b1, w2lp,
                    h_ref, g_ref, rc_ref):
  cnt = jnp.maximum(cnt2[0] + cnt2[1], 1.0)
  rc = 1.0 / cnt
  agg = (acc[0] + acc[1]) * rc
  h = (jnp.dot(agg, w1l[...], preferred_element_type=jnp.float32)
       + b1[...]
       + jnp.dot(x[...], w1r[...], preferred_element_type=jnp.float32))
  h = jnp.maximum(h, 0.0)
  h_ref[...] = h
  g_ref[...] = jnp.dot(h, w2lp[...], preferred_element_type=jnp.float32)
  rc_ref[...] = rc


def _layer2_tc_body(acc, rc, h, w2rp, b2p, o_ref):
  z = ((acc[0] + acc[1]) * rc[...]
       + jnp.dot(h[...], w2rp[...], preferred_element_type=jnp.float32)
       + b2p[...])
  o_ref[...] = jax.nn.sigmoid(z)[:, :D_OUT]


def _row_spec(d):
  return pl.BlockSpec((BN, d), lambda i: (i, 0))


def _full_spec(r, c):
  return pl.BlockSpec((r, c), lambda i: (0, 0))


def kernel(x, edge_index, W1_l, W1_r, b1, W2_l, W2_r, b2):
  idx = jnp.transpose(edge_index.reshape(2, NW, NCHUNK, K), (1, 2, 0, 3))

  acc1, cnt = _segment_sum_sc(x, idx, with_cnt=True)

  w2lp = jnp.pad(W2_l, ((0, 0), (0, D_IN - D_OUT)))
  grid = (N // BN,)
  h, g, rc = pl.pallas_call(
      _layer1_tc_body,
      grid=grid,
      in_specs=[
          pl.BlockSpec((NC, BN, D_IN), lambda i: (0, i, 0)),
          pl.BlockSpec((NC, BN, 1), lambda i: (0, i, 0)),
          _row_spec(D_IN),
          _full_spec(D_IN, D_HID), _full_spec(D_IN, D_HID),
          _full_spec(1, D_HID),
          _full_spec(D_HID, D_IN),
      ],
      out_specs=[_row_spec(D_HID), _row_spec(D_IN), _row_spec(1)],
      out_shape=[
          jax.ShapeDtypeStruct((N, D_HID), jnp.float32),
          jax.ShapeDtypeStruct((N, D_IN), jnp.float32),
          jax.ShapeDtypeStruct((N, 1), jnp.float32),
      ],
  )(acc1, cnt.reshape(NC, N, 1), x, W1_l, W1_r, b1[None, :], w2lp)

  acc2, _ = _segment_sum_sc(g, idx, with_cnt=False)

  w2rp = jnp.pad(W2_r, ((0, 0), (0, D_IN - D_OUT)))
  b2p = jnp.pad(b2, (0, D_IN - D_OUT))
  o = pl.pallas_call(
      _layer2_tc_body,
      grid=grid,
      in_specs=[
          pl.BlockSpec((NC, BN, D_IN), lambda i: (0, i, 0)),
          _row_spec(1),
          _row_spec(D_HID),
          _full_spec(D_HID, D_IN),
          _full_spec(1, D_IN),
      ],
      out_specs=_row_spec(D_OUT),
      out_shape=jax.ShapeDtypeStruct((N, D_OUT), jnp.float32),
  )(acc2, rc, h, w2rp, b2p[None, :])

  return o
